# baseline (speedup 1.0000x reference)
import jax
import jax.numpy as jnp
from jax import lax
from jax.experimental import pallas as pl
from jax.experimental.pallas import tpu as pltpu
from jax.experimental.pallas import tpu_sc as plsc

VOCAB_SIZE = 100000
EMBED_DIM = 64
MAX_LEN = 200
BATCH = 4096
SEQ_LEN = 200

NUM_WORKERS = 32
SEQ_PER_W = BATCH // NUM_WORKERS
HALF = SEQ_LEN // 2
NB = 6
AHEAD = 4
MAIN = (SEQ_PER_W // NB) * NB


def _emb_body(x_hbm, tok_hbm, pos_hbm, out_hbm, pos_sh, idx_v,
              b0, b1, b2, b3, b4, b5,
              g0, g1, g2, g3, g4, g5,
              s0, s1, s2, s3, s4, s5,
              p0, p1, p2, p3, p4, p5):
    c = lax.axis_index("c")
    s = lax.axis_index("s")
    wid = s * 2 + c
    row0 = wid * SEQ_PER_W * SEQ_LEN

    bufs = (b0, b1, b2, b3, b4, b5)
    gsems = (g0, g1, g2, g3, g4, g5)
    ssems = (s0, s1, s2, s3, s4, s5)
    psems = (p0, p1, p2, p3, p4, p5)

    @pl.when(s == 0)
    def _():
        pltpu.sync_copy(pos_hbm, pos_sh)

    plsc.subcore_barrier()

    def prefill(p):
        pltpu.async_copy(pos_sh, bufs[p], psems[p])

    def wait_prefill(p):
        pltpu.make_async_copy(pos_sh, bufs[p], psems[p]).wait()

    def gather_add(g, p):
        pltpu.async_copy(tok_hbm.at[idx_v.at[g, 0]],
                         bufs[p].at[pl.ds(0, HALF)], gsems[p], add=True)
        pltpu.async_copy(tok_hbm.at[idx_v.at[g, 1]],
                         bufs[p].at[pl.ds(HALF, HALF)], gsems[p], add=True)

    def wait_gather(g, p):
        pltpu.make_async_copy(tok_hbm.at[idx_v.at[g, 0]],
                              bufs[p].at[pl.ds(0, HALF)], gsems[p]).wait()
        pltpu.make_async_copy(tok_hbm.at[idx_v.at[g, 1]],
                              bufs[p].at[pl.ds(HALF, HALF)], gsems[p]).wait()

    def store(g, p):
        pltpu.async_copy(
            bufs[p], out_hbm.at[pl.ds(row0 + g * SEQ_LEN, SEQ_LEN)], ssems[p])

    def wait_store(g, p):
        pltpu.make_async_copy(
            bufs[p], out_hbm.at[pl.ds(row0 + g * SEQ_LEN, SEQ_LEN)],
            ssems[p]).wait()

    pltpu.sync_copy(x_hbm.at[pl.ds(wid * SEQ_PER_W, SEQ_PER_W)], idx_v)
    for k in range(NB - 1):
        prefill(k)
    for k in range(AHEAD):
        wait_prefill(k)
        gather_add(k, k)

    def slot(g, p, static=False):
        q = (p + NB - 1) % NB

        def _drain():
            wait_store(g - 1, q)

        def _refill():
            prefill(q)

        def _launch():
            r = (p + AHEAD) % NB
            wait_prefill(r)
            gather_add(g + AHEAD, r)

        if static:
            if g >= 1:
                _drain()
            if g + NB - 1 < SEQ_PER_W:
                _refill()
            if g + AHEAD < SEQ_PER_W:
                _launch()
        else:
            pl.when(g >= 1)(_drain)
            pl.when(g + NB - 1 < SEQ_PER_W)(_refill)
            pl.when(g + AHEAD < SEQ_PER_W)(_launch)
        wait_gather(g, p)
        store(g, p)

    def step(t, carry):
        for k in range(NB):
            slot(NB * t + k, k)
        return carry

    lax.fori_loop(0, MAIN // NB, step, 0)

    for g in range(MAIN, SEQ_PER_W):
        slot(g, g % NB, static=True)

    wait_store(SEQ_PER_W - 1, (SEQ_PER_W - 1) % NB)


@jax.jit
def kernel(x, token_table, pos_table):
    x3 = x.astype(jnp.int32).reshape(BATCH, 2, HALF)
    mesh = plsc.VectorSubcoreMesh(core_axis_name="c", subcore_axis_name="s")
    out_flat = pl.kernel(
        _emb_body,
        out_type=jax.ShapeDtypeStruct((BATCH * SEQ_LEN, EMBED_DIM),
                                      jnp.float32),
        mesh=mesh,
        scratch_types=[
            pltpu.VMEM_SHARED((MAX_LEN, EMBED_DIM), jnp.float32),
            pltpu.VMEM((SEQ_PER_W, 2, HALF), jnp.int32),
        ] + [pltpu.VMEM((SEQ_LEN, EMBED_DIM), jnp.float32)] * NB
          + [pltpu.SemaphoreType.DMA] * (3 * NB),
        compiler_params=pltpu.CompilerParams(use_tc_tiling_on_sc=False),
    )(x3, token_table, pos_table)
    return out_flat.reshape(BATCH, SEQ_LEN, EMBED_DIM)


# speedup vs baseline: 1.0115x; 1.0115x over previous
import jax
import jax.numpy as jnp
from jax import lax
from jax.experimental import pallas as pl
from jax.experimental.pallas import tpu as pltpu
from jax.experimental.pallas import tpu_sc as plsc

VOCAB_SIZE = 100000
EMBED_DIM = 64
MAX_LEN = 200
BATCH = 4096
SEQ_LEN = 200

NUM_WORKERS = 32
SEQ_PER_SLOT = 2
SLOT_ROWS = SEQ_PER_SLOT * SEQ_LEN
NSLOT = BATCH // (NUM_WORKERS * SEQ_PER_SLOT)
HALF = SEQ_LEN // 2
NSTR = SLOT_ROWS // HALF
NB = 3
MAIN = (NSLOT // NB) * NB


def _emb_body(x_hbm, tok_hbm, pos_hbm, out_hbm, pos_sh, idx_v,
              b0, b1, b2,
              g0, g1, g2, s0, s1, s2, p0, p1, p2):
    c = lax.axis_index("c")
    s = lax.axis_index("s")
    wid = s * 2 + c
    row0 = wid * NSLOT * SLOT_ROWS

    bufs = (b0, b1, b2)
    gsems = (g0, g1, g2)
    ssems = (s0, s1, s2)
    psems = (p0, p1, p2)

    @pl.when(s == 0)
    def _():
        pltpu.sync_copy(pos_hbm, pos_sh.at[pl.ds(0, SEQ_LEN)])
        pltpu.sync_copy(pos_hbm, pos_sh.at[pl.ds(SEQ_LEN, SEQ_LEN)])

    plsc.subcore_barrier()

    def prefill(p):
        pltpu.async_copy(pos_sh, bufs[p], psems[p])

    def wait_prefill(p):
        pltpu.make_async_copy(pos_sh, bufs[p], psems[p]).wait()

    def gather_add(g, p):
        for j in range(NSTR):
            pltpu.async_copy(tok_hbm.at[idx_v.at[g, j]],
                             bufs[p].at[pl.ds(j * HALF, HALF)],
                             gsems[p], add=True)

    def wait_gather(g, p):
        for j in range(NSTR):
            pltpu.make_async_copy(tok_hbm.at[idx_v.at[g, j]],
                                  bufs[p].at[pl.ds(j * HALF, HALF)],
                                  gsems[p]).wait()

    def store(g, p):
        pltpu.async_copy(
            bufs[p], out_hbm.at[pl.ds(row0 + g * SLOT_ROWS, SLOT_ROWS)],
            ssems[p])

    def wait_store(g, p):
        pltpu.make_async_copy(
            bufs[p], out_hbm.at[pl.ds(row0 + g * SLOT_ROWS, SLOT_ROWS)],
            ssems[p]).wait()

    pltpu.sync_copy(x_hbm.at[pl.ds(wid * NSLOT, NSLOT)], idx_v)
    prefill(0)
    prefill(1)
    wait_prefill(0)
    gather_add(0, 0)

    def slot(g, p, static=False):
        q = (p + 2) % NB

        def _drain():
            wait_store(g - 1, q)

        def _refill():
            prefill(q)

        def _launch():
            r = (p + 1) % NB
            wait_prefill(r)
            gather_add(g + 1, r)

        if static:
            if g >= 1:
                _drain()
            if g + 2 < NSLOT:
                _refill()
            if g + 1 < NSLOT:
                _launch()
        else:
            pl.when(g >= 1)(_drain)
            pl.when(g + 2 < NSLOT)(_refill)
            pl.when(g + 1 < NSLOT)(_launch)
        wait_gather(g, p)
        store(g, p)

    def step(t, carry):
        for k in range(NB):
            slot(NB * t + k, k)
        return carry

    lax.fori_loop(0, MAIN // NB, step, 0)

    for g in range(MAIN, NSLOT):
        slot(g, g % NB, static=True)

    wait_store(NSLOT - 1, (NSLOT - 1) % NB)


@jax.jit
def kernel(x, token_table, pos_table):
    x3 = x.astype(jnp.int32).reshape(BATCH // SEQ_PER_SLOT, NSTR, HALF)
    mesh = plsc.VectorSubcoreMesh(core_axis_name="c", subcore_axis_name="s")
    out_flat = pl.kernel(
        _emb_body,
        out_type=jax.ShapeDtypeStruct((BATCH * SEQ_LEN, EMBED_DIM),
                                      jnp.float32),
        mesh=mesh,
        scratch_types=[
            pltpu.VMEM_SHARED((SLOT_ROWS, EMBED_DIM), jnp.float32),
            pltpu.VMEM((NSLOT, NSTR, HALF), jnp.int32),
        ] + [pltpu.VMEM((SLOT_ROWS, EMBED_DIM), jnp.float32)] * NB
          + [pltpu.SemaphoreType.DMA] * (3 * NB),
        compiler_params=pltpu.CompilerParams(use_tc_tiling_on_sc=False),
    )(x3, token_table, pos_table)
    return out_flat.reshape(BATCH, SEQ_LEN, EMBED_DIM)
